# Initial kernel scaffold; baseline (speedup 1.0000x reference)
import jax
import jax.numpy as jnp
from jax.experimental import pallas as pl


def kernel(noise, index):
    raise NotImplementedError("write your pallas kernel here")


# speedup vs baseline: 1.9533x; 1.9533x over previous
import functools

import jax
import jax.numpy as jnp
from jax import lax
from jax.experimental import pallas as pl
from jax.experimental.pallas import tpu as pltpu
from jax.experimental.pallas import tpu_sc as plsc

N = 800000
DIM = 64
SEGS = 50000

NC = 2
NS = 16
HALF = DIM // NC
UNIT = 128
UNITS = N // UNIT
UNITS_PER_SUB = -(-UNITS // NS)

SEG_BLK = 400
NBLK = SEGS // SEG_BLK
BLK_PER_SUB = -(-NBLK // NS)

_mesh = plsc.VectorSubcoreMesh(core_axis_name="c", subcore_axis_name="s")


@functools.partial(
    pl.kernel,
    out_type=jax.ShapeDtypeStruct((N, DIM), jnp.float32),
    mesh=_mesh,
    compiler_params=pltpu.CompilerParams(
        use_tc_tiling_on_sc=False, needs_layout_passes=False),
    scratch_types=[
        pltpu.VMEM((UNIT,), jnp.int32),
        pltpu.VMEM((UNIT, HALF), jnp.float32),
        pltpu.VMEM((UNIT, HALF), jnp.float32),
        pltpu.VMEM((UNIT,), jnp.float32),
        pltpu.VMEM((SEG_BLK, HALF), jnp.float32),
        pltpu.VMEM((SEG_BLK,), jnp.float32),
        pltpu.VMEM_SHARED((SEGS, HALF), jnp.float32),
        pltpu.VMEM_SHARED((SEGS,), jnp.float32),
    ],
)
def _center_sc(noise_hbm, idx_hbm, out_hbm,
               idx_v, noise_v, mean_v, ones_v, sums_v, counts_v,
               sum_sh, cnt_sh):
    c = lax.axis_index("c")
    s = lax.axis_index("s")
    col0 = pl.multiple_of(c * HALF, HALF)

    zeros16 = jnp.zeros((16,), jnp.float32)
    ones16 = jnp.ones((16,), jnp.float32)

    for i in range(UNIT // 16):
        ones_v[pl.ds(i * 16, 16)] = ones16

    def _zero_srow(r, _):
        for h in range(HALF // 16):
            sums_v[r, pl.ds(h * 16, 16)] = zeros16
        return 0
    lax.fori_loop(0, SEG_BLK, _zero_srow, 0)
    for g in range(SEG_BLK // 16):
        counts_v[pl.ds(g * 16, 16)] = zeros16

    for j in range(BLK_PER_SUB):
        b = s * BLK_PER_SUB + j

        @pl.when(b < NBLK)
        def _():
            base = pl.multiple_of(b * SEG_BLK, 8)
            pltpu.sync_copy(sums_v, sum_sh.at[pl.ds(base, SEG_BLK)])
            pltpu.sync_copy(counts_v, cnt_sh.at[pl.ds(base, SEG_BLK)])

    plsc.subcore_barrier()

    def _p1(t, _):
        m = s + t * NS

        @pl.when(m < UNITS)
        def _():
            row0 = pl.multiple_of(m * UNIT, UNIT)
            pltpu.sync_copy(idx_hbm.at[pl.ds(row0, UNIT)], idx_v)
            pltpu.sync_copy(
                noise_hbm.at[pl.ds(row0, UNIT), pl.ds(col0, HALF)], noise_v)
            pltpu.sync_copy(noise_v, sum_sh.at[idx_v], add=True)
            pltpu.sync_copy(ones_v, cnt_sh.at[idx_v], add=True)
        return 0
    lax.fori_loop(0, UNITS_PER_SUB, _p1, 0)

    plsc.subcore_barrier()

    iota16 = lax.iota(jnp.int32, 16)
    for j in range(BLK_PER_SUB):
        b = s * BLK_PER_SUB + j

        @pl.when(b < NBLK)
        def _():
            base = pl.multiple_of(b * SEG_BLK, 8)
            pltpu.sync_copy(sum_sh.at[pl.ds(base, SEG_BLK)], sums_v)
            pltpu.sync_copy(cnt_sh.at[pl.ds(base, SEG_BLK)], counts_v)

            def _norm16(g, _):
                cnt = counts_v[pl.ds(g * 16, 16)]
                inv = 1.0 / jnp.maximum(cnt, 1.0)
                o_idx = g * 16 + iota16
                for col in range(HALF):
                    ci = jnp.full((16,), col, jnp.int32)
                    v = plsc.load_gather(sums_v, [o_idx, ci])
                    plsc.store_scatter(sums_v, [o_idx, ci], v * inv)
                return 0
            lax.fori_loop(0, SEG_BLK // 16, _norm16, 0)
            pltpu.sync_copy(sums_v, sum_sh.at[pl.ds(base, SEG_BLK)])

    plsc.subcore_barrier()

    def _p3(t, _):
        m = s + t * NS

        @pl.when(m < UNITS)
        def _():
            row0 = pl.multiple_of(m * UNIT, UNIT)
            pltpu.sync_copy(idx_hbm.at[pl.ds(row0, UNIT)], idx_v)
            pltpu.sync_copy(
                noise_hbm.at[pl.ds(row0, UNIT), pl.ds(col0, HALF)], noise_v)
            pltpu.sync_copy(sum_sh.at[idx_v], mean_v)

            def _sub(r, _):
                for h in range(HALF // 16):
                    sl = pl.ds(h * 16, 16)
                    noise_v[r, sl] = noise_v[r, sl] - mean_v[r, sl]
                return 0
            lax.fori_loop(0, UNIT, _sub, 0)
            pltpu.sync_copy(
                noise_v, out_hbm.at[pl.ds(row0, UNIT), pl.ds(col0, HALF)])
        return 0
    lax.fori_loop(0, UNITS_PER_SUB, _p3, 0)


def kernel(noise, index):
    return _center_sc(noise, index.astype(jnp.int32))


# speedup vs baseline: 3.1038x; 1.5890x over previous
import functools

import jax
import jax.numpy as jnp
from jax import lax
from jax.experimental import pallas as pl
from jax.experimental.pallas import tpu as pltpu
from jax.experimental.pallas import tpu_sc as plsc

N = 800000
DIM = 64
SEGS = 50000

NC = 2
NS = 16
HALF = DIM // NC
UNIT = 128
UNITS = N // UNIT
T_MAX = -(-UNITS // NS)

SEG_BLK = 80
NBLK = SEGS // SEG_BLK
BLK_PER_SUB = -(-NBLK // NS)

_mesh = plsc.VectorSubcoreMesh(core_axis_name="c", subcore_axis_name="s")


@functools.partial(
    pl.kernel,
    out_type=jax.ShapeDtypeStruct((N, DIM), jnp.float32),
    mesh=_mesh,
    compiler_params=pltpu.CompilerParams(
        use_tc_tiling_on_sc=False, needs_layout_passes=False),
    scratch_types=[
        pltpu.VMEM((2, UNIT), jnp.int32),
        pltpu.VMEM((2, UNIT, HALF), jnp.float32),
        pltpu.VMEM((2, UNIT, HALF), jnp.float32),
        pltpu.VMEM((UNIT,), jnp.float32),
        pltpu.VMEM((SEG_BLK, HALF), jnp.float32),
        pltpu.VMEM((SEG_BLK,), jnp.float32),
        pltpu.VMEM_SHARED((SEGS, HALF), jnp.float32),
        pltpu.VMEM_SHARED((SEGS,), jnp.float32),
        pltpu.SemaphoreType.DMA,
        pltpu.SemaphoreType.DMA,
        pltpu.SemaphoreType.DMA,
        pltpu.SemaphoreType.DMA,
        pltpu.SemaphoreType.DMA,
    ],
)
def _center_sc(noise_hbm, idx_hbm, out_hbm,
               idx_v, noise_v, mean_v, ones_v, sums_v, counts_v,
               sum_sh, cnt_sh, isem0, isem1, nsem0, nsem1, ssem):
    c = lax.axis_index("c")
    s = lax.axis_index("s")
    col0 = pl.multiple_of(c * HALF, HALF)
    isems = (isem0, isem1)
    nsems = (nsem0, nsem1)

    zeros16 = jnp.zeros((16,), jnp.float32)
    ones16 = jnp.ones((16,), jnp.float32)

    def unit_id(t):
        return s + t * NS

    def load_copies(t, sl):
        m = unit_id(t)
        row0 = pl.multiple_of(m * UNIT, UNIT)
        icopy = pltpu.make_async_copy(idx_hbm.at[m], idx_v.at[sl], isems[sl])
        ncopy = pltpu.make_async_copy(
            noise_hbm.at[pl.ds(row0, UNIT), pl.ds(col0, HALF)],
            noise_v.at[sl], nsems[sl])
        return icopy, ncopy

    def start_loads(t, sl):
        @pl.when(unit_id(t) < UNITS)
        def _():
            icopy, ncopy = load_copies(t, sl)
            icopy.start()
            ncopy.start()

    def wait_loads(t, sl):
        icopy, ncopy = load_copies(t, sl)
        icopy.wait()
        ncopy.wait()

    for i in range(UNIT // 16):
        ones_v[pl.ds(i * 16, 16)] = ones16

    def _zero_srow(r, _):
        for h in range(HALF // 16):
            sums_v[r, pl.ds(h * 16, 16)] = zeros16
        return 0
    lax.fori_loop(0, SEG_BLK, _zero_srow, 0)
    for g in range(SEG_BLK // 16):
        counts_v[pl.ds(g * 16, 16)] = zeros16

    def _zblk(j, _):
        b = s * BLK_PER_SUB + j

        @pl.when(b < NBLK)
        def _():
            base = pl.multiple_of(b * SEG_BLK, 8)
            pltpu.sync_copy(sums_v, sum_sh.at[pl.ds(base, SEG_BLK)])
            pltpu.sync_copy(counts_v, cnt_sh.at[pl.ds(base, SEG_BLK)])
        return 0
    lax.fori_loop(0, BLK_PER_SUB, _zblk, 0)

    plsc.subcore_barrier()

    start_loads(0, 0)
    start_loads(1, 1)

    def _p1(tt, _):
        for sl in (0, 1):
            t = 2 * tt + sl

            @pl.when(unit_id(t) < UNITS)
            def _():
                wait_loads(t, sl)
                a = pltpu.make_async_copy(
                    noise_v.at[sl], sum_sh.at[idx_v.at[sl]], ssem)
                b = pltpu.make_async_copy(
                    ones_v, cnt_sh.at[idx_v.at[sl]], ssem)
                a.start(add=True)
                b.start(add=True)
                a.wait()
                b.wait()
            start_loads(t + 2, sl)
        return 0
    lax.fori_loop(0, (T_MAX + 1) // 2, _p1, 0)

    plsc.subcore_barrier()

    iota16 = lax.iota(jnp.int32, 16)

    def _nblk(j, _):
        b = s * BLK_PER_SUB + j

        @pl.when(b < NBLK)
        def _():
            base = pl.multiple_of(b * SEG_BLK, 8)
            pltpu.sync_copy(sum_sh.at[pl.ds(base, SEG_BLK)], sums_v)
            pltpu.sync_copy(cnt_sh.at[pl.ds(base, SEG_BLK)], counts_v)

            def _norm16(g, _):
                cnt = counts_v[pl.ds(g * 16, 16)]
                inv = 1.0 / jnp.maximum(cnt, 1.0)
                o_idx = g * 16 + iota16
                for col in range(HALF):
                    ci = jnp.full((16,), col, jnp.int32)
                    v = plsc.load_gather(sums_v, [o_idx, ci])
                    plsc.store_scatter(sums_v, [o_idx, ci], v * inv)
                return 0
            lax.fori_loop(0, SEG_BLK // 16, _norm16, 0)
            pltpu.sync_copy(sums_v, sum_sh.at[pl.ds(base, SEG_BLK)])
        return 0
    lax.fori_loop(0, BLK_PER_SUB, _nblk, 0)

    plsc.subcore_barrier()

    start_loads(0, 0)
    start_loads(1, 1)

    def _p3(tt, _):
        for sl in (0, 1):
            t = 2 * tt + sl

            @pl.when(unit_id(t) < UNITS)
            def _():
                m = unit_id(t)
                row0 = pl.multiple_of(m * UNIT, UNIT)
                wait_loads(t, sl)
                g = pltpu.make_async_copy(
                    sum_sh.at[idx_v.at[sl]], mean_v.at[sl], ssem)
                g.start()
                g.wait()

                def _sub(r4, _):
                    for rr in range(4):
                        r = r4 * 4 + rr
                        for h in range(HALF // 16):
                            d = pl.ds(h * 16, 16)
                            noise_v[sl, r, d] = (
                                noise_v[sl, r, d] - mean_v[sl, r, d])
                    return 0
                lax.fori_loop(0, UNIT // 4, _sub, 0)
                pltpu.sync_copy(
                    noise_v.at[sl],
                    out_hbm.at[pl.ds(row0, UNIT), pl.ds(col0, HALF)])
            start_loads(t + 2, sl)
        return 0
    lax.fori_loop(0, (T_MAX + 1) // 2, _p3, 0)


def kernel(noise, index):
    idx2d = index.astype(jnp.int32).reshape(N // UNIT, UNIT)
    return _center_sc(noise, idx2d)


# speedup vs baseline: 3.1576x; 1.0173x over previous
import functools

import jax
import jax.numpy as jnp
from jax import lax
from jax.experimental import pallas as pl
from jax.experimental.pallas import tpu as pltpu
from jax.experimental.pallas import tpu_sc as plsc

N = 800000
DIM = 64
SEGS = 50000

NC = 2
NS = 16
HALF = DIM // NC
UNIT = 128
UNITS = N // UNIT
T_MAX = -(-UNITS // NS)

SEG_BLK = 80
NBLK = SEGS // SEG_BLK
BLK_PER_SUB = -(-NBLK // NS)

_mesh = plsc.VectorSubcoreMesh(core_axis_name="c", subcore_axis_name="s")


@functools.partial(
    pl.kernel,
    out_type=jax.ShapeDtypeStruct((N, DIM), jnp.float32),
    mesh=_mesh,
    compiler_params=pltpu.CompilerParams(
        use_tc_tiling_on_sc=False, needs_layout_passes=False),
    scratch_types=[
        pltpu.VMEM((2, UNIT), jnp.int32),
        pltpu.VMEM((2, UNIT, HALF), jnp.float32),
        pltpu.VMEM((2, UNIT, HALF), jnp.float32),
        pltpu.VMEM((UNIT,), jnp.float32),
        pltpu.VMEM((SEG_BLK, HALF), jnp.float32),
        pltpu.VMEM((SEG_BLK,), jnp.float32),
        pltpu.VMEM_SHARED((SEGS, HALF), jnp.float32),
        pltpu.VMEM_SHARED((SEGS,), jnp.float32),
        pltpu.SemaphoreType.DMA,
        pltpu.SemaphoreType.DMA,
        pltpu.SemaphoreType.DMA,
        pltpu.SemaphoreType.DMA,
        pltpu.SemaphoreType.DMA,
        pltpu.SemaphoreType.DMA,
        pltpu.SemaphoreType.DMA,
    ],
)
def _center_sc(noise_hbm, idx_hbm, out_hbm,
               idx_v, noise_v, mean_v, ones_v, sums_v, counts_v,
               sum_sh, cnt_sh, isem0, isem1, nsem0, nsem1, ssem,
               wsem0, wsem1):
    c = lax.axis_index("c")
    s = lax.axis_index("s")
    col0 = pl.multiple_of(c * HALF, HALF)
    isems = (isem0, isem1)
    nsems = (nsem0, nsem1)
    wsems = (wsem0, wsem1)

    zeros16 = jnp.zeros((16,), jnp.float32)
    ones16 = jnp.ones((16,), jnp.float32)

    def unit_id(t):
        return s + t * NS

    def load_copies(t, sl):
        m = unit_id(t)
        row0 = pl.multiple_of(m * UNIT, UNIT)
        icopy = pltpu.make_async_copy(idx_hbm.at[m], idx_v.at[sl], isems[sl])
        ncopy = pltpu.make_async_copy(
            noise_hbm.at[pl.ds(row0, UNIT), pl.ds(col0, HALF)],
            noise_v.at[sl], nsems[sl])
        return icopy, ncopy

    def start_loads(t, sl):
        @pl.when(unit_id(t) < UNITS)
        def _():
            icopy, ncopy = load_copies(t, sl)
            icopy.start()
            ncopy.start()

    def wait_loads(t, sl):
        icopy, ncopy = load_copies(t, sl)
        icopy.wait()
        ncopy.wait()

    for i in range(UNIT // 16):
        ones_v[pl.ds(i * 16, 16)] = ones16

    def _zero_srow(r, _):
        for h in range(HALF // 16):
            sums_v[r, pl.ds(h * 16, 16)] = zeros16
        return 0
    lax.fori_loop(0, SEG_BLK, _zero_srow, 0)
    for g in range(SEG_BLK // 16):
        counts_v[pl.ds(g * 16, 16)] = zeros16

    def _zblk(j, _):
        b = s * BLK_PER_SUB + j

        @pl.when(b < NBLK)
        def _():
            base = pl.multiple_of(b * SEG_BLK, 8)
            pltpu.sync_copy(sums_v, sum_sh.at[pl.ds(base, SEG_BLK)])
            pltpu.sync_copy(counts_v, cnt_sh.at[pl.ds(base, SEG_BLK)])
        return 0
    lax.fori_loop(0, BLK_PER_SUB, _zblk, 0)

    plsc.subcore_barrier()

    start_loads(0, 0)
    start_loads(1, 1)

    def _p1(tt, _):
        for sl in (0, 1):
            t = 2 * tt + sl

            @pl.when(unit_id(t) < UNITS)
            def _():
                wait_loads(t, sl)
                a = pltpu.make_async_copy(
                    noise_v.at[sl], sum_sh.at[idx_v.at[sl]], ssem)
                b = pltpu.make_async_copy(
                    ones_v, cnt_sh.at[idx_v.at[sl]], ssem)
                a.start(add=True)
                b.start(add=True)
                a.wait()
                b.wait()
            start_loads(t + 2, sl)
        return 0
    lax.fori_loop(0, (T_MAX + 1) // 2, _p1, 0)

    plsc.subcore_barrier()

    iota16 = lax.iota(jnp.int32, 16)

    def _nblk(j, _):
        b = s * BLK_PER_SUB + j

        @pl.when(b < NBLK)
        def _():
            base = pl.multiple_of(b * SEG_BLK, 8)
            pltpu.sync_copy(sum_sh.at[pl.ds(base, SEG_BLK)], sums_v)
            pltpu.sync_copy(cnt_sh.at[pl.ds(base, SEG_BLK)], counts_v)

            def _norm16(g, _):
                cnt = counts_v[pl.ds(g * 16, 16)]
                inv = 1.0 / jnp.maximum(cnt, 1.0)
                o_idx = g * 16 + iota16
                for col in range(HALF):
                    ci = jnp.full((16,), col, jnp.int32)
                    v = plsc.load_gather(sums_v, [o_idx, ci])
                    plsc.store_scatter(sums_v, [o_idx, ci], v * inv)
                return 0
            lax.fori_loop(0, SEG_BLK // 16, _norm16, 0)
            pltpu.sync_copy(sums_v, sum_sh.at[pl.ds(base, SEG_BLK)])
        return 0
    lax.fori_loop(0, BLK_PER_SUB, _nblk, 0)

    plsc.subcore_barrier()

    def out_copy(t, sl):
        m = unit_id(t)
        row0 = pl.multiple_of(m * UNIT, UNIT)
        return pltpu.make_async_copy(
            mean_v.at[sl],
            out_hbm.at[pl.ds(row0, UNIT), pl.ds(col0, HALF)], wsems[sl])

    start_loads(0, 0)
    start_loads(1, 1)

    def _p3(tt, _):
        for sl in (0, 1):
            t = 2 * tt + sl
            m_prev = unit_id(t - 2)

            @pl.when(jnp.logical_and(0 <= m_prev, m_prev < UNITS))
            def _():
                out_copy(t - 2, sl).wait()

            @pl.when(unit_id(t) < UNITS)
            def _():
                wait_loads(t, sl)
                g = pltpu.make_async_copy(
                    sum_sh.at[idx_v.at[sl]], mean_v.at[sl], ssem)
                g.start()
                g.wait()

                @plsc.parallel_loop(0, UNIT, 4, unroll=2)
                def _sub(r4):
                    for rr in range(4):
                        r = r4 + rr
                        for h in range(HALF // 16):
                            d = pl.ds(h * 16, 16)
                            mean_v[sl, r, d] = (
                                noise_v[sl, r, d] - mean_v[sl, r, d])

                out_copy(t, sl).start()
                start_loads(t + 2, sl)
        return 0
    lax.fori_loop(0, (T_MAX + 1) // 2 + 1, _p3, 0)


def kernel(noise, index):
    idx2d = index.astype(jnp.int32).reshape(N // UNIT, UNIT)
    return _center_sc(noise, idx2d)
